# baseline (speedup 1.0000x reference)
import functools

import jax
import jax.numpy as jnp
from jax import lax
from jax.experimental import pallas as pl
from jax.experimental.pallas import tpu as pltpu
from jax.experimental.pallas import tpu_sc as plsc

N_NODES = 100000
N_EDGES = 3200000
D_IN = 128
D_OUT = 16
D_HALF = 8

NPAD = 100352
EPAD = 3211264
G_TOTAL = EPAD // 128
G_PER_TILE32 = G_TOTAL // 32
G_PER_TILE16 = G_TOTAL // 16
SUPER32 = G_PER_TILE32 // 16
PAIRS16 = G_PER_TILE16 // 32
ROWS_PER_TILE = NPAD // 16
NPT = NPAD // 32

_mesh = plsc.VectorSubcoreMesh(
    core_axis_name="c", subcore_axis_name="s", num_cores=2, num_subcores=16)

_sc_params = pltpu.CompilerParams(use_tc_tiling_on_sc=False)
_sc_params_nl = pltpu.CompilerParams(
    use_tc_tiling_on_sc=False, needs_layout_passes=False)


@functools.partial(
    pl.kernel,
    out_type=(jax.ShapeDtypeStruct((NPAD,), jnp.float32),
              jax.ShapeDtypeStruct((NPAD,), jnp.float32)),
    mesh=_mesh,
    scratch_types=[
        pltpu.VMEM((2, 16, 128), jnp.int32),
        pltpu.VMEM((128,), jnp.float32),
        pltpu.VMEM_SHARED((NPAD,), jnp.float32),
        pltpu.SemaphoreType.DMA,
    ],
    compiler_params=_sc_params,
)
def _deg_kernel(colg_hbm, ones_hbm, zeros_hbm, deg0_hbm, deg1_hbm,
                colidx_v, ones_v, deg_sh, sem):
    cid = lax.axis_index("c")
    sid = lax.axis_index("s")
    wid = sid * 2 + cid
    sl = pl.ds(sid * ROWS_PER_TILE, ROWS_PER_TILE)
    pltpu.sync_copy(zeros_hbm.at[sl], deg_sh.at[sl])
    pltpu.sync_copy(ones_hbm, ones_v)
    plsc.subcore_barrier()

    base_g = wid * G_PER_TILE32

    def stage(c, par):
        pltpu.sync_copy(colg_hbm.at[pl.ds(base_g + c * 16, 16)],
                        colidx_v.at[par])

    def fire(par):
        for j in range(16):
            pltpu.async_copy(ones_v, deg_sh.at[colidx_v.at[par].at[j]],
                             sem, add=True)

    def drain(par):
        pltpu.make_async_copy(colg_hbm.at[pl.ds(0, 16)],
                              colidx_v.at[par], sem).wait()

    stage(0, 0)

    def body(i, carry):
        a = 2 * i
        fire(0)
        stage(a + 1, 1)
        drain(0)
        fire(1)
        stage(a + 2, 0)
        drain(1)
        return carry

    lax.fori_loop(0, (SUPER32 - 1) // 2, body, 0)
    fire(0)
    drain(0)
    plsc.subcore_barrier()

    @pl.when(cid == 0)
    def _():
        pltpu.sync_copy(deg_sh.at[sl], deg0_hbm.at[sl])

    @pl.when(cid == 1)
    def _():
        pltpu.sync_copy(deg_sh.at[sl], deg1_hbm.at[sl])


def _rsqrt16(v):
    i = plsc.bitcast(v, jnp.int32)
    i = 0x5F3759DF - lax.shift_right_arithmetic(i, 1)
    r = plsc.bitcast(i, jnp.float32)
    for _ in range(3):
        r = r * (1.5 - 0.5 * v * r * r)
    return r


@functools.partial(
    pl.kernel,
    out_type=(jax.ShapeDtypeStruct((NPAD * D_HALF,), jnp.float32),
              jax.ShapeDtypeStruct((NPAD * D_HALF,), jnp.float32),
              jax.ShapeDtypeStruct((NPAD,), jnp.float32)),
    mesh=_mesh,
    scratch_types=[
        pltpu.VMEM((NPT,), jnp.float32),
        pltpu.VMEM((NPT,), jnp.float32),
        pltpu.VMEM((NPT * D_OUT,), jnp.float32),
        pltpu.VMEM((NPT * D_HALF,), jnp.float32),
        pltpu.VMEM((NPT * D_HALF,), jnp.float32),
    ],
    compiler_params=_sc_params_nl,
)
def _scale_kernel(deg0_hbm, deg1_hbm, xf_hbm, y0f_hbm, y1f_hbm, disf_hbm,
                  d0_v, d1_v, xt_v, y0_v, y1_v):
    cid = lax.axis_index("c")
    sid = lax.axis_index("s")
    tid = sid * 2 + cid
    nsl = pl.ds(tid * NPT, NPT)
    pltpu.sync_copy(deg0_hbm.at[nsl], d0_v)
    pltpu.sync_copy(deg1_hbm.at[nsl], d1_v)
    pltpu.sync_copy(xf_hbm.at[pl.ds(tid * NPT * D_OUT, NPT * D_OUT)], xt_v)

    def rs_body(k, carry):
        v = d0_v[pl.ds(k * 16, 16)] + d1_v[pl.ds(k * 16, 16)] + 1.0
        d0_v[pl.ds(k * 16, 16)] = _rsqrt16(v)
        return carry

    lax.fori_loop(0, NPT // 16, rs_body, 0)

    iota = lax.iota(jnp.int32, 16)
    half = lax.iota(jnp.int32, 16) // 8
    pat0 = (iota & 7) + 16 * half

    def sc_body(m, carry):
        i0 = pat0 + m * 32
        s = plsc.load_gather(d0_v, [half + m * 2])
        g0 = plsc.load_gather(xt_v, [i0])
        g1 = plsc.load_gather(xt_v, [i0 + 8])
        y0_v[pl.ds(m * 16, 16)] = g0 * s
        y1_v[pl.ds(m * 16, 16)] = g1 * s
        return carry

    lax.fori_loop(0, NPT // 2, sc_body, 0)

    pltpu.sync_copy(y0_v, y0f_hbm.at[pl.ds(tid * NPT * D_HALF, NPT * D_HALF)])
    pltpu.sync_copy(y1_v, y1f_hbm.at[pl.ds(tid * NPT * D_HALF, NPT * D_HALF)])
    pltpu.sync_copy(d0_v, disf_hbm.at[nsl])


@functools.partial(
    pl.kernel,
    out_type=(jax.ShapeDtypeStruct((NPAD, D_HALF), jnp.float32),
              jax.ShapeDtypeStruct((NPAD, D_HALF), jnp.float32)),
    mesh=_mesh,
    scratch_types=[
        pltpu.VMEM((2, 16, 128), jnp.int32),
        pltpu.VMEM((2, 16, 128), jnp.int32),
        pltpu.VMEM((2, 2048, D_HALF), jnp.float32),
        pltpu.VMEM_SHARED((NPAD, D_HALF), jnp.float32),
        pltpu.SemaphoreType.DMA,
        pltpu.SemaphoreType.DMA,
    ],
    compiler_params=_sc_params,
)
def _prop_kernel(rowg_hbm, colg_hbm, y0_hbm, y1_hbm, acc0_hbm, acc1_hbm,
                 rowidx_v, colidx_v, ybuf, acc_sh, sem_g, sem_s):
    cid = lax.axis_index("c")
    sid = lax.axis_index("s")
    sl = pl.ds(sid * ROWS_PER_TILE, ROWS_PER_TILE)
    base_g = sid * G_PER_TILE16

    def run(y_src, acc_out):
        pltpu.sync_copy(y_src.at[sl], acc_sh.at[sl])
        plsc.subcore_barrier()

        def stage(c, par):
            g0 = base_g + c * 16
            pltpu.sync_copy(rowg_hbm.at[pl.ds(g0, 16)], rowidx_v.at[par])
            pltpu.sync_copy(colg_hbm.at[pl.ds(g0, 16)], colidx_v.at[par])

        def fire_g(par):
            for j in range(16):
                pltpu.async_copy(y_src.at[rowidx_v.at[par].at[j]],
                                 ybuf.at[par].at[pl.ds(j * 128, 128)], sem_g)

        def fire_s(par):
            for j in range(16):
                pltpu.async_copy(ybuf.at[par].at[pl.ds(j * 128, 128)],
                                 acc_sh.at[colidx_v.at[par].at[j]], sem_s,
                                 add=True)

        def drain(sem, par):
            pltpu.make_async_copy(y_src.at[pl.ds(0, 2048)],
                                  ybuf.at[par], sem).wait()

        n_batch = G_PER_TILE16 // 16
        stage(0, 0)
        fire_g(0)

        def body(i, carry):
            pl.when(i >= 1)(lambda: drain(sem_s, 1))
            drain(sem_g, 0)
            stage(2 * i + 1, 1)
            fire_g(1)
            fire_s(0)
            drain(sem_s, 0)
            drain(sem_g, 1)

            @pl.when(i < n_batch // 2 - 1)
            def _():
                stage(2 * i + 2, 0)
                fire_g(0)

            fire_s(1)
            return carry

        lax.fori_loop(0, n_batch // 2, body, 0)
        drain(sem_s, 1)
        plsc.subcore_barrier()
        pltpu.sync_copy(acc_sh.at[sl], acc_out.at[sl])

    @pl.when(cid == 0)
    def _():
        run(y0_hbm, acc0_hbm)

    @pl.when(cid == 1)
    def _():
        run(y1_hbm, acc1_hbm)


@functools.partial(
    pl.kernel,
    out_type=jax.ShapeDtypeStruct((NPAD * D_OUT,), jnp.float32),
    mesh=_mesh,
    scratch_types=[
        pltpu.VMEM((NPT,), jnp.float32),
        pltpu.VMEM((NPT * D_HALF,), jnp.float32),
        pltpu.VMEM((NPT * D_HALF,), jnp.float32),
        pltpu.VMEM((NPT * D_OUT,), jnp.float32),
    ],
    compiler_params=_sc_params_nl,
)
def _finish_kernel(acc0f_hbm, acc1f_hbm, disf_hbm, outf_hbm,
                   dis_v, a0_v, a1_v, o_v):
    cid = lax.axis_index("c")
    sid = lax.axis_index("s")
    tid = sid * 2 + cid
    pltpu.sync_copy(disf_hbm.at[pl.ds(tid * NPT, NPT)], dis_v)
    pltpu.sync_copy(acc0f_hbm.at[pl.ds(tid * NPT * D_HALF, NPT * D_HALF)],
                    a0_v)
    pltpu.sync_copy(acc1f_hbm.at[pl.ds(tid * NPT * D_HALF, NPT * D_HALF)],
                    a1_v)

    iota = lax.iota(jnp.int32, 16)
    half = lax.iota(jnp.int32, 16) // 8
    pat0 = (iota & 7) + 16 * half

    def body(m, carry):
        s = plsc.load_gather(dis_v, [half + m * 2])
        v0 = a0_v[pl.ds(m * 16, 16)]
        v1 = a1_v[pl.ds(m * 16, 16)]
        o0 = jnp.maximum(v0 * s, 0.0)
        o1 = jnp.maximum(v1 * s, 0.0)
        i0 = pat0 + m * 32
        plsc.store_scatter(o_v, [i0], o0)
        plsc.store_scatter(o_v, [i0 + 8], o1)
        return carry

    lax.fori_loop(0, NPT // 2, body, 0)
    pltpu.sync_copy(o_v, outf_hbm.at[pl.ds(tid * NPT * D_OUT, NPT * D_OUT)])


def _linear_body(a_ref, w_ref, b_ref, x_ref):
    x = lax.dot_general(a_ref[...], w_ref[...],
                        (((1,), (1,)), ((), ())),
                        preferred_element_type=jnp.float32)
    x_ref[...] = x + b_ref[...]


def kernel(atom, edge_index, W, b):
    row = edge_index[0]
    col = edge_index[1]
    npad_e = EPAD - N_EDGES
    rowg = jnp.concatenate(
        [row, jnp.zeros((npad_e,), jnp.int32)]).reshape(G_TOTAL, 128)
    colg = jnp.concatenate(
        [col, jnp.full((npad_e,), N_NODES, jnp.int32)]).reshape(G_TOTAL, 128)

    ones128 = jnp.ones((128,), jnp.float32)
    zeros_n = jnp.zeros((NPAD,), jnp.float32)
    deg0, deg1 = _deg_kernel(colg, ones128, zeros_n)

    b2 = b.reshape(1, D_OUT)
    grid = NPAD // 1024
    x = pl.pallas_call(
        _linear_body,
        grid=(grid,),
        in_specs=[
            pl.BlockSpec((1024, D_IN), lambda i: (i, 0)),
            pl.BlockSpec((D_OUT, D_IN), lambda i: (0, 0)),
            pl.BlockSpec((1, D_OUT), lambda i: (0, 0)),
        ],
        out_specs=pl.BlockSpec((1024, D_OUT), lambda i: (i, 0)),
        out_shape=jax.ShapeDtypeStruct((NPAD, D_OUT), jnp.float32),
    )(atom, W, b2)

    xf = x.reshape(NPAD * D_OUT)
    y0f, y1f, disf = _scale_kernel(deg0, deg1, xf)
    y0 = y0f.reshape(NPAD, D_HALF)
    y1 = y1f.reshape(NPAD, D_HALF)

    acc0, acc1 = _prop_kernel(rowg, colg, y0, y1)

    outf = _finish_kernel(acc0.reshape(NPAD * D_HALF),
                          acc1.reshape(NPAD * D_HALF), disf)
    return outf.reshape(NPAD, D_OUT)[:N_NODES]


# speedup vs baseline: 1.1270x; 1.1270x over previous
import functools

import jax
import jax.numpy as jnp
from jax import lax
from jax.experimental import pallas as pl
from jax.experimental.pallas import tpu as pltpu
from jax.experimental.pallas import tpu_sc as plsc

N_NODES = 100000
N_EDGES = 3200000
D_IN = 128
D_OUT = 16
D_HALF = 8

NPAD = 100352
EPAD = 3211264
G_TOTAL = EPAD // 128
G_PER_TILE32 = G_TOTAL // 32
G_PER_TILE16 = G_TOTAL // 16
SUPER32 = G_PER_TILE32 // 16
PAIRS16 = G_PER_TILE16 // 32
ROWS_PER_TILE = NPAD // 16
NPT = NPAD // 32

_mesh = plsc.VectorSubcoreMesh(
    core_axis_name="c", subcore_axis_name="s", num_cores=2, num_subcores=16)

_sc_params = pltpu.CompilerParams(use_tc_tiling_on_sc=False)
_sc_params_nl = pltpu.CompilerParams(
    use_tc_tiling_on_sc=False, needs_layout_passes=False)


@functools.partial(
    pl.kernel,
    out_type=(jax.ShapeDtypeStruct((NPAD,), jnp.float32),
              jax.ShapeDtypeStruct((NPAD,), jnp.float32)),
    mesh=_mesh,
    scratch_types=[
        pltpu.VMEM((2, 16, 128), jnp.int32),
        pltpu.VMEM((128,), jnp.float32),
        pltpu.VMEM_SHARED((NPAD,), jnp.float32),
        pltpu.SemaphoreType.DMA,
    ],
    compiler_params=_sc_params,
)
def _deg_kernel(colg_hbm, ones_hbm, zeros_hbm, deg0_hbm, deg1_hbm,
                colidx_v, ones_v, deg_sh, sem):
    cid = lax.axis_index("c")
    sid = lax.axis_index("s")
    wid = sid * 2 + cid
    sl = pl.ds(sid * ROWS_PER_TILE, ROWS_PER_TILE)
    pltpu.sync_copy(zeros_hbm.at[sl], deg_sh.at[sl])
    pltpu.sync_copy(ones_hbm, ones_v)
    plsc.subcore_barrier()

    base_g = wid * G_PER_TILE32

    def stage(c, par):
        pltpu.sync_copy(colg_hbm.at[pl.ds(base_g + c * 16, 16)],
                        colidx_v.at[par])

    def fire(par):
        for j in range(16):
            pltpu.async_copy(ones_v, deg_sh.at[colidx_v.at[par].at[j]],
                             sem, add=True)

    def drain(par):
        pltpu.make_async_copy(colg_hbm.at[pl.ds(0, 16)],
                              colidx_v.at[par], sem).wait()

    stage(0, 0)

    def body(i, carry):
        a = 2 * i
        fire(0)
        stage(a + 1, 1)
        drain(0)
        fire(1)
        stage(a + 2, 0)
        drain(1)
        return carry

    lax.fori_loop(0, (SUPER32 - 1) // 2, body, 0)
    fire(0)
    drain(0)
    plsc.subcore_barrier()

    @pl.when(cid == 0)
    def _():
        pltpu.sync_copy(deg_sh.at[sl], deg0_hbm.at[sl])

    @pl.when(cid == 1)
    def _():
        pltpu.sync_copy(deg_sh.at[sl], deg1_hbm.at[sl])


def _rsqrt16(v):
    i = plsc.bitcast(v, jnp.int32)
    i = 0x5F3759DF - lax.shift_right_arithmetic(i, 1)
    r = plsc.bitcast(i, jnp.float32)
    for _ in range(3):
        r = r * (1.5 - 0.5 * v * r * r)
    return r


@functools.partial(
    pl.kernel,
    out_type=(jax.ShapeDtypeStruct((NPAD * D_HALF,), jnp.float32),
              jax.ShapeDtypeStruct((NPAD * D_HALF,), jnp.float32),
              jax.ShapeDtypeStruct((NPAD,), jnp.float32)),
    mesh=_mesh,
    scratch_types=[
        pltpu.VMEM((NPT,), jnp.float32),
        pltpu.VMEM((NPT,), jnp.float32),
        pltpu.VMEM((NPT * D_OUT,), jnp.float32),
        pltpu.VMEM((NPT * D_HALF,), jnp.float32),
        pltpu.VMEM((NPT * D_HALF,), jnp.float32),
    ],
    compiler_params=_sc_params_nl,
)
def _scale_kernel(deg0_hbm, deg1_hbm, xf_hbm, y0f_hbm, y1f_hbm, disf_hbm,
                  d0_v, d1_v, xt_v, y0_v, y1_v):
    cid = lax.axis_index("c")
    sid = lax.axis_index("s")
    tid = sid * 2 + cid
    nsl = pl.ds(tid * NPT, NPT)
    pltpu.sync_copy(deg0_hbm.at[nsl], d0_v)
    pltpu.sync_copy(deg1_hbm.at[nsl], d1_v)
    pltpu.sync_copy(xf_hbm.at[pl.ds(tid * NPT * D_OUT, NPT * D_OUT)], xt_v)

    def rs_body(k, carry):
        v = d0_v[pl.ds(k * 16, 16)] + d1_v[pl.ds(k * 16, 16)] + 1.0
        d0_v[pl.ds(k * 16, 16)] = _rsqrt16(v)
        return carry

    lax.fori_loop(0, NPT // 16, rs_body, 0)

    iota = lax.iota(jnp.int32, 16)
    half = lax.iota(jnp.int32, 16) // 8
    pat0 = (iota & 7) + 16 * half

    def sc_body(m, carry):
        i0 = pat0 + m * 32
        s = plsc.load_gather(d0_v, [half + m * 2])
        g0 = plsc.load_gather(xt_v, [i0])
        g1 = plsc.load_gather(xt_v, [i0 + 8])
        y0_v[pl.ds(m * 16, 16)] = g0 * s
        y1_v[pl.ds(m * 16, 16)] = g1 * s
        return carry

    lax.fori_loop(0, NPT // 2, sc_body, 0)

    pltpu.sync_copy(y0_v, y0f_hbm.at[pl.ds(tid * NPT * D_HALF, NPT * D_HALF)])
    pltpu.sync_copy(y1_v, y1f_hbm.at[pl.ds(tid * NPT * D_HALF, NPT * D_HALF)])
    pltpu.sync_copy(d0_v, disf_hbm.at[nsl])


@functools.partial(
    pl.kernel,
    out_type=(jax.ShapeDtypeStruct((NPAD, D_HALF), jnp.float32),
              jax.ShapeDtypeStruct((NPAD, D_HALF), jnp.float32)),
    mesh=_mesh,
    scratch_types=[
        pltpu.VMEM((2, 16, 128), jnp.int32),
        pltpu.VMEM((2, 16, 128), jnp.int32),
        pltpu.VMEM((2, 2048, D_HALF), jnp.float32),
        pltpu.VMEM_SHARED((NPAD, D_HALF), jnp.float32),
        pltpu.SemaphoreType.DMA,
        pltpu.SemaphoreType.DMA,
    ],
    compiler_params=_sc_params,
)
def _prop_kernel(rowg_hbm, colg_hbm, y0_hbm, y1_hbm, acc0_hbm, acc1_hbm,
                 rowidx_v, colidx_v, ybuf, acc_sh, sem_g, sem_s):
    cid = lax.axis_index("c")
    sid = lax.axis_index("s")
    sl = pl.ds(sid * ROWS_PER_TILE, ROWS_PER_TILE)
    base_g = sid * G_PER_TILE16

    def run(y_src, acc_out):
        pltpu.sync_copy(y_src.at[sl], acc_sh.at[sl])
        plsc.subcore_barrier()

        def stage(c, par):
            g0 = base_g + c * 16
            pltpu.sync_copy(rowg_hbm.at[pl.ds(g0, 16)], rowidx_v.at[par])
            pltpu.sync_copy(colg_hbm.at[pl.ds(g0, 16)], colidx_v.at[par])

        def fire_g(par):
            for j in range(16):
                pltpu.async_copy(y_src.at[rowidx_v.at[par].at[j]],
                                 ybuf.at[par].at[pl.ds(j * 128, 128)], sem_g)

        def fire_s(par):
            for j in range(16):
                pltpu.async_copy(ybuf.at[par].at[pl.ds(j * 128, 128)],
                                 acc_sh.at[colidx_v.at[par].at[j]], sem_s,
                                 add=True)

        def gwait(par):
            pltpu.make_async_copy(y_src.at[pl.ds(0, 2048)],
                                  ybuf.at[par], sem_g).wait()

        def swait(par):
            pltpu.make_async_copy(y_src.at[pl.ds(0, 2048)],
                                  ybuf.at[par], sem_s).wait()

        n_batch = G_PER_TILE16 // 16
        stage(0, 0)

        def body(i, carry):
            fire_g(0)
            stage(2 * i + 1, 1)
            gwait(0)
            fire_s(0)
            fire_g(1)
            swait(0)

            @pl.when(i < n_batch // 2 - 1)
            def _():
                stage(2 * i + 2, 0)

            gwait(1)
            fire_s(1)
            swait(1)
            return carry

        lax.fori_loop(0, n_batch // 2, body, 0)
        plsc.subcore_barrier()
        pltpu.sync_copy(acc_sh.at[sl], acc_out.at[sl])

    @pl.when(cid == 0)
    def _():
        run(y0_hbm, acc0_hbm)

    @pl.when(cid == 1)
    def _():
        run(y1_hbm, acc1_hbm)


@functools.partial(
    pl.kernel,
    out_type=jax.ShapeDtypeStruct((NPAD * D_OUT,), jnp.float32),
    mesh=_mesh,
    scratch_types=[
        pltpu.VMEM((NPT,), jnp.float32),
        pltpu.VMEM((NPT * D_HALF,), jnp.float32),
        pltpu.VMEM((NPT * D_HALF,), jnp.float32),
        pltpu.VMEM((NPT * D_OUT,), jnp.float32),
    ],
    compiler_params=_sc_params_nl,
)
def _finish_kernel(acc0f_hbm, acc1f_hbm, disf_hbm, outf_hbm,
                   dis_v, a0_v, a1_v, o_v):
    cid = lax.axis_index("c")
    sid = lax.axis_index("s")
    tid = sid * 2 + cid
    pltpu.sync_copy(disf_hbm.at[pl.ds(tid * NPT, NPT)], dis_v)
    pltpu.sync_copy(acc0f_hbm.at[pl.ds(tid * NPT * D_HALF, NPT * D_HALF)],
                    a0_v)
    pltpu.sync_copy(acc1f_hbm.at[pl.ds(tid * NPT * D_HALF, NPT * D_HALF)],
                    a1_v)

    iota = lax.iota(jnp.int32, 16)
    half = lax.iota(jnp.int32, 16) // 8
    pat0 = (iota & 7) + 16 * half

    def body(m, carry):
        s = plsc.load_gather(dis_v, [half + m * 2])
        v0 = a0_v[pl.ds(m * 16, 16)]
        v1 = a1_v[pl.ds(m * 16, 16)]
        o0 = jnp.maximum(v0 * s, 0.0)
        o1 = jnp.maximum(v1 * s, 0.0)
        i0 = pat0 + m * 32
        plsc.store_scatter(o_v, [i0], o0)
        plsc.store_scatter(o_v, [i0 + 8], o1)
        return carry

    lax.fori_loop(0, NPT // 2, body, 0)
    pltpu.sync_copy(o_v, outf_hbm.at[pl.ds(tid * NPT * D_OUT, NPT * D_OUT)])


def _linear_body(a_ref, w_ref, b_ref, x_ref):
    x = lax.dot_general(a_ref[...], w_ref[...],
                        (((1,), (1,)), ((), ())),
                        preferred_element_type=jnp.float32)
    x_ref[...] = x + b_ref[...]


def kernel(atom, edge_index, W, b):
    row = edge_index[0]
    col = edge_index[1]
    npad_e = EPAD - N_EDGES
    rowg = jnp.concatenate(
        [row, jnp.zeros((npad_e,), jnp.int32)]).reshape(G_TOTAL, 128)
    colg = jnp.concatenate(
        [col, jnp.full((npad_e,), N_NODES, jnp.int32)]).reshape(G_TOTAL, 128)

    ones128 = jnp.ones((128,), jnp.float32)
    zeros_n = jnp.zeros((NPAD,), jnp.float32)
    deg0, deg1 = _deg_kernel(colg, ones128, zeros_n)

    b2 = b.reshape(1, D_OUT)
    grid = NPAD // 1024
    x = pl.pallas_call(
        _linear_body,
        grid=(grid,),
        in_specs=[
            pl.BlockSpec((1024, D_IN), lambda i: (i, 0)),
            pl.BlockSpec((D_OUT, D_IN), lambda i: (0, 0)),
            pl.BlockSpec((1, D_OUT), lambda i: (0, 0)),
        ],
        out_specs=pl.BlockSpec((1024, D_OUT), lambda i: (i, 0)),
        out_shape=jax.ShapeDtypeStruct((NPAD, D_OUT), jnp.float32),
    )(atom, W, b2)

    xf = x.reshape(NPAD * D_OUT)
    y0f, y1f, disf = _scale_kernel(deg0, deg1, xf)
    y0 = y0f.reshape(NPAD, D_HALF)
    y1 = y1f.reshape(NPAD, D_HALF)

    acc0, acc1 = _prop_kernel(rowg, colg, y0, y1)

    outf = _finish_kernel(acc0.reshape(NPAD * D_HALF),
                          acc1.reshape(NPAD * D_HALF), disf)
    return outf.reshape(NPAD, D_OUT)[:N_NODES]


# speedup vs baseline: 1.1360x; 1.0080x over previous
import functools

import jax
import jax.numpy as jnp
from jax import lax
from jax.experimental import pallas as pl
from jax.experimental.pallas import tpu as pltpu
from jax.experimental.pallas import tpu_sc as plsc

N_NODES = 100000
N_EDGES = 3200000
D_IN = 128
D_OUT = 16
D_HALF = 8

NPAD = 100352
EPAD = 3211264
G_TOTAL = EPAD // 128
G_PER_TILE32 = G_TOTAL // 32
G_PER_TILE16 = G_TOTAL // 16
SUPER32 = G_PER_TILE32 // 16
PAIRS16 = G_PER_TILE16 // 32
ROWS_PER_TILE = NPAD // 16
NPT = NPAD // 32

_mesh = plsc.VectorSubcoreMesh(
    core_axis_name="c", subcore_axis_name="s", num_cores=2, num_subcores=16)

_sc_params = pltpu.CompilerParams(use_tc_tiling_on_sc=False)
_sc_params_nl = pltpu.CompilerParams(
    use_tc_tiling_on_sc=False, needs_layout_passes=False)


@functools.partial(
    pl.kernel,
    out_type=(jax.ShapeDtypeStruct((NPAD,), jnp.float32),
              jax.ShapeDtypeStruct((NPAD,), jnp.float32)),
    mesh=_mesh,
    scratch_types=[
        pltpu.VMEM((2, 16, 128), jnp.int32),
        pltpu.VMEM((128,), jnp.float32),
        pltpu.VMEM_SHARED((NPAD,), jnp.float32),
        pltpu.SemaphoreType.DMA,
    ],
    compiler_params=_sc_params,
)
def _deg_kernel(colg_hbm, ones_hbm, zeros_hbm, deg0_hbm, deg1_hbm,
                colidx_v, ones_v, deg_sh, sem):
    cid = lax.axis_index("c")
    sid = lax.axis_index("s")
    wid = sid * 2 + cid
    sl = pl.ds(sid * ROWS_PER_TILE, ROWS_PER_TILE)
    pltpu.sync_copy(zeros_hbm.at[sl], deg_sh.at[sl])
    pltpu.sync_copy(ones_hbm, ones_v)
    plsc.subcore_barrier()

    base_g = wid * G_PER_TILE32

    def stage(c, par):
        pltpu.sync_copy(colg_hbm.at[pl.ds(base_g + c * 16, 16)],
                        colidx_v.at[par])

    def fire(par):
        for j in range(16):
            pltpu.async_copy(ones_v, deg_sh.at[colidx_v.at[par].at[j]],
                             sem, add=True)

    def drain(par):
        pltpu.make_async_copy(colg_hbm.at[pl.ds(0, 16)],
                              colidx_v.at[par], sem).wait()

    stage(0, 0)

    def body(i, carry):
        a = 2 * i
        fire(0)
        stage(a + 1, 1)
        drain(0)
        fire(1)
        stage(a + 2, 0)
        drain(1)
        return carry

    lax.fori_loop(0, (SUPER32 - 1) // 2, body, 0)
    fire(0)
    drain(0)
    plsc.subcore_barrier()

    @pl.when(cid == 0)
    def _():
        pltpu.sync_copy(deg_sh.at[sl], deg0_hbm.at[sl])

    @pl.when(cid == 1)
    def _():
        pltpu.sync_copy(deg_sh.at[sl], deg1_hbm.at[sl])


def _rsqrt16(v):
    i = plsc.bitcast(v, jnp.int32)
    i = 0x5F3759DF - lax.shift_right_arithmetic(i, 1)
    r = plsc.bitcast(i, jnp.float32)
    for _ in range(3):
        r = r * (1.5 - 0.5 * v * r * r)
    return r


@functools.partial(
    pl.kernel,
    out_type=(jax.ShapeDtypeStruct((NPAD * D_HALF,), jnp.float32),
              jax.ShapeDtypeStruct((NPAD * D_HALF,), jnp.float32),
              jax.ShapeDtypeStruct((NPAD,), jnp.float32)),
    mesh=_mesh,
    scratch_types=[
        pltpu.VMEM((NPT,), jnp.float32),
        pltpu.VMEM((NPT,), jnp.float32),
        pltpu.VMEM((NPT * D_OUT,), jnp.float32),
        pltpu.VMEM((NPT * D_HALF,), jnp.float32),
        pltpu.VMEM((NPT * D_HALF,), jnp.float32),
    ],
    compiler_params=_sc_params_nl,
)
def _scale_kernel(deg0_hbm, deg1_hbm, xf_hbm, y0f_hbm, y1f_hbm, disf_hbm,
                  d0_v, d1_v, xt_v, y0_v, y1_v):
    cid = lax.axis_index("c")
    sid = lax.axis_index("s")
    tid = sid * 2 + cid
    nsl = pl.ds(tid * NPT, NPT)
    pltpu.sync_copy(deg0_hbm.at[nsl], d0_v)
    pltpu.sync_copy(deg1_hbm.at[nsl], d1_v)
    pltpu.sync_copy(xf_hbm.at[pl.ds(tid * NPT * D_OUT, NPT * D_OUT)], xt_v)

    def rs_body(k, carry):
        v = d0_v[pl.ds(k * 16, 16)] + d1_v[pl.ds(k * 16, 16)] + 1.0
        d0_v[pl.ds(k * 16, 16)] = _rsqrt16(v)
        return carry

    lax.fori_loop(0, NPT // 16, rs_body, 0)

    iota = lax.iota(jnp.int32, 16)
    half = lax.iota(jnp.int32, 16) // 8
    pat0 = (iota & 7) + 16 * half

    def sc_body(m, carry):
        i0 = pat0 + m * 32
        s = plsc.load_gather(d0_v, [half + m * 2])
        g0 = plsc.load_gather(xt_v, [i0])
        g1 = plsc.load_gather(xt_v, [i0 + 8])
        y0_v[pl.ds(m * 16, 16)] = g0 * s
        y1_v[pl.ds(m * 16, 16)] = g1 * s
        return carry

    lax.fori_loop(0, NPT // 2, sc_body, 0)

    pltpu.sync_copy(y0_v, y0f_hbm.at[pl.ds(tid * NPT * D_HALF, NPT * D_HALF)])
    pltpu.sync_copy(y1_v, y1f_hbm.at[pl.ds(tid * NPT * D_HALF, NPT * D_HALF)])
    pltpu.sync_copy(d0_v, disf_hbm.at[nsl])


@functools.partial(
    pl.kernel,
    out_type=(jax.ShapeDtypeStruct((NPAD, D_HALF), jnp.float32),
              jax.ShapeDtypeStruct((NPAD, D_HALF), jnp.float32)),
    mesh=_mesh,
    scratch_types=[
        pltpu.VMEM((2, 16, 128), jnp.int32),
        pltpu.VMEM((2, 16, 128), jnp.int32),
        pltpu.VMEM((2, 2048, D_HALF), jnp.float32),
        pltpu.VMEM_SHARED((NPAD, D_HALF), jnp.float32),
        pltpu.SemaphoreType.DMA,
        pltpu.SemaphoreType.DMA,
    ],
    compiler_params=_sc_params,
)
def _prop_kernel(rowg_hbm, colg_hbm, y0_hbm, y1_hbm, acc0_hbm, acc1_hbm,
                 rowidx_v, colidx_v, ybuf, acc_sh, sem_g, sem_s):
    cid = lax.axis_index("c")
    sid = lax.axis_index("s")
    sl = pl.ds(sid * ROWS_PER_TILE, ROWS_PER_TILE)
    base_g = sid * G_PER_TILE16

    def run(y_src, acc_out):
        pltpu.sync_copy(y_src.at[sl], acc_sh.at[sl])
        plsc.subcore_barrier()

        def stage(c, par):
            g0 = base_g + c * 16
            pltpu.sync_copy(rowg_hbm.at[pl.ds(g0, 16)], rowidx_v.at[par])
            pltpu.sync_copy(colg_hbm.at[pl.ds(g0, 16)], colidx_v.at[par])

        def fire_g(par):
            for j in range(16):
                pltpu.async_copy(y_src.at[rowidx_v.at[par].at[j]],
                                 ybuf.at[par].at[pl.ds(j * 128, 128)], sem_g)

        def fire_s(par):
            for j in range(16):
                pltpu.async_copy(ybuf.at[par].at[pl.ds(j * 128, 128)],
                                 acc_sh.at[colidx_v.at[par].at[j]], sem_s,
                                 add=True)

        def gwait(par):
            pltpu.make_async_copy(y_src.at[pl.ds(0, 2048)],
                                  ybuf.at[par], sem_g).wait()

        def swait(par):
            pltpu.make_async_copy(y_src.at[pl.ds(0, 2048)],
                                  ybuf.at[par], sem_s).wait()

        n_batch = G_PER_TILE16 // 16
        stage(0, 0)

        def body(i, carry):
            fire_g(0)
            stage(2 * i + 1, 1)
            gwait(0)
            fire_s(0)
            fire_g(1)
            swait(0)

            @pl.when(i < n_batch // 2 - 1)
            def _():
                stage(2 * i + 2, 0)

            gwait(1)
            fire_s(1)
            swait(1)
            return carry

        lax.fori_loop(0, n_batch // 2, body, 0)
        plsc.subcore_barrier()
        pltpu.sync_copy(acc_sh.at[sl], acc_out.at[sl])

    @pl.when(cid == 0)
    def _():
        run(y0_hbm, acc0_hbm)

    @pl.when(cid == 1)
    def _():
        run(y1_hbm, acc1_hbm)


@functools.partial(
    pl.kernel,
    out_type=jax.ShapeDtypeStruct((N_NODES, D_OUT), jnp.float32),
    mesh=_mesh,
    scratch_types=[
        pltpu.VMEM((NPT,), jnp.float32),
        pltpu.VMEM((NPT * D_HALF,), jnp.float32),
        pltpu.VMEM((NPT * D_HALF,), jnp.float32),
        pltpu.VMEM((NPT, D_OUT), jnp.float32),
    ],
    compiler_params=_sc_params_nl,
)
def _finish_kernel(acc0f_hbm, acc1f_hbm, disf_hbm, out_hbm,
                   dis_v, a0_v, a1_v, o_v):
    cid = lax.axis_index("c")
    sid = lax.axis_index("s")
    tid = sid * 2 + cid
    pltpu.sync_copy(disf_hbm.at[pl.ds(tid * NPT, NPT)], dis_v)
    pltpu.sync_copy(acc0f_hbm.at[pl.ds(tid * NPT * D_HALF, NPT * D_HALF)],
                    a0_v)
    pltpu.sync_copy(acc1f_hbm.at[pl.ds(tid * NPT * D_HALF, NPT * D_HALF)],
                    a1_v)

    iota = lax.iota(jnp.int32, 16)
    half = lax.iota(jnp.int32, 16) // 8
    col0 = iota & 7

    def body(m, carry):
        s = plsc.load_gather(dis_v, [half + m * 2])
        v0 = a0_v[pl.ds(m * 16, 16)]
        v1 = a1_v[pl.ds(m * 16, 16)]
        o0 = jnp.maximum(v0 * s, 0.0)
        o1 = jnp.maximum(v1 * s, 0.0)
        r = half + m * 2
        plsc.store_scatter(o_v, [r, col0], o0)
        plsc.store_scatter(o_v, [r, col0 + 8], o1)
        return carry

    lax.fori_loop(0, NPT // 2, body, 0)
    last = N_NODES - 31 * NPT

    @pl.when(tid < 31)
    def _():
        pltpu.sync_copy(o_v, out_hbm.at[pl.ds(tid * NPT, NPT)])

    @pl.when(tid == 31)
    def _():
        pltpu.sync_copy(o_v.at[pl.ds(0, last)],
                        out_hbm.at[pl.ds(31 * NPT, last)])


def _linear_body(a_ref, w_ref, b_ref, x_ref):
    x = lax.dot_general(a_ref[...], w_ref[...],
                        (((1,), (1,)), ((), ())),
                        preferred_element_type=jnp.float32)
    x_ref[...] = x + b_ref[...]


def kernel(atom, edge_index, W, b):
    row = edge_index[0]
    col = edge_index[1]
    npad_e = EPAD - N_EDGES
    rowg = jnp.concatenate(
        [row, jnp.zeros((npad_e,), jnp.int32)]).reshape(G_TOTAL, 128)
    colg = jnp.concatenate(
        [col, jnp.full((npad_e,), N_NODES, jnp.int32)]).reshape(G_TOTAL, 128)

    ones128 = jnp.ones((128,), jnp.float32)
    zeros_n = jnp.zeros((NPAD,), jnp.float32)
    deg0, deg1 = _deg_kernel(colg, ones128, zeros_n)

    b2 = b.reshape(1, D_OUT)
    grid = NPAD // 1024
    x = pl.pallas_call(
        _linear_body,
        grid=(grid,),
        in_specs=[
            pl.BlockSpec((1024, D_IN), lambda i: (i, 0)),
            pl.BlockSpec((D_OUT, D_IN), lambda i: (0, 0)),
            pl.BlockSpec((1, D_OUT), lambda i: (0, 0)),
        ],
        out_specs=pl.BlockSpec((1024, D_OUT), lambda i: (i, 0)),
        out_shape=jax.ShapeDtypeStruct((NPAD, D_OUT), jnp.float32),
    )(atom, W, b2)

    xf = x.reshape(NPAD * D_OUT)
    y0f, y1f, disf = _scale_kernel(deg0, deg1, xf)
    y0 = y0f.reshape(NPAD, D_HALF)
    y1 = y1f.reshape(NPAD, D_HALF)

    acc0, acc1 = _prop_kernel(rowg, colg, y0, y1)

    return _finish_kernel(acc0.reshape(NPAD * D_HALF),
                          acc1.reshape(NPAD * D_HALF), disf)
